# Initial kernel scaffold; baseline (speedup 1.0000x reference)
import functools

import jax
import jax.numpy as jnp
from jax import lax
from jax.experimental import pallas as pl
from jax.experimental.pallas import tpu as pltpu
from jax.experimental.pallas import tpu_sc as plsc

B, S, NCAT, V, ED, CD, DM = 4096, 50, 26, 100000, 32, 16, 128
T = B * S
N = T * NCAT
RPT = 128
NW = 32
BLOCKS = N // RPT
BPW = BLOCKS // NW
K = 13
OUTER = BPW // K


def _sc_gather(tables_flat, idx_blocks):
    mesh = plsc.VectorSubcoreMesh(core_axis_name="c", subcore_axis_name="s")

    @functools.partial(
        pl.kernel,
        out_type=jax.ShapeDtypeStruct((N, ED), jnp.float32),
        mesh=mesh,
        scratch_types=[
            pltpu.VMEM((K, RPT), jnp.int32),
            pltpu.VMEM((K * RPT, ED), jnp.float32),
            pltpu.SemaphoreType.DMA,
        ],
    )
    def gather_kernel(tbl_hbm, idx_hbm, out_hbm, idx_v, rows_v, sem):
        wid = lax.axis_index("s") * 2 + lax.axis_index("c")
        wbase = wid * BPW

        def body(g, carry):
            blk0 = wbase + g * K
            pltpu.sync_copy(idx_hbm.at[pl.ds(blk0, K)], idx_v)
            copies = []
            for j in range(K):
                copies.append(
                    pltpu.async_copy(
                        tbl_hbm.at[idx_v.at[j]],
                        rows_v.at[pl.ds(j * RPT, RPT)],
                        sem,
                    )
                )
            for c in copies:
                c.wait()
            pltpu.sync_copy(rows_v, out_hbm.at[pl.ds(blk0 * RPT, K * RPT)])
            return carry

        lax.fori_loop(0, OUTER, body, 0)

    return gather_kernel(tables_flat, idx_blocks)


def _proj_body(x_ref, n_ref, m_ref, wc_ref, wn_ref, b_ref, o_ref):
    acc = jnp.dot(x_ref[...], wc_ref[...], preferred_element_type=jnp.float32)
    acc += jnp.dot(n_ref[...] * m_ref[...], wn_ref[...],
                   preferred_element_type=jnp.float32)
    o_ref[...] = acc + b_ref[...]


def _tc_project(cat_repr, num_inputs, num_mask, Wc, Wn, b2):
    BT = 2048
    grid = (T // BT,)
    return pl.pallas_call(
        _proj_body,
        grid=grid,
        in_specs=[
            pl.BlockSpec((BT, NCAT * ED), lambda i: (i, 0)),
            pl.BlockSpec((BT, CD), lambda i: (i, 0)),
            pl.BlockSpec((BT, CD), lambda i: (i, 0)),
            pl.BlockSpec((NCAT * ED, DM), lambda i: (0, 0)),
            pl.BlockSpec((CD, DM), lambda i: (0, 0)),
            pl.BlockSpec((1, DM), lambda i: (0, 0)),
        ],
        out_specs=pl.BlockSpec((BT, DM), lambda i: (i, 0)),
        out_shape=jax.ShapeDtypeStruct((T, DM), jnp.float32),
    )(cat_repr, num_inputs, num_mask, Wc, Wn, b2)


def kernel(cat_inputs, cat_mask, num_inputs, num_mask, tables, W, b):
    tables_flat = tables.reshape(NCAT * V, ED)
    offs = (jnp.arange(NCAT, dtype=jnp.int32) * V)[None, None, :]
    idx_blocks = (cat_inputs + offs).reshape(BLOCKS, RPT)

    rows = _sc_gather(tables_flat, idx_blocks)
    cat_repr = rows.reshape(T, NCAT * ED)

    out = _tc_project(
        cat_repr,
        num_inputs.reshape(T, CD),
        num_mask.reshape(T, CD),
        W[: NCAT * ED],
        W[NCAT * ED :],
        b.reshape(1, DM),
    )
    return out.reshape(B, S, DM)


# speedup vs baseline: 7.1152x; 7.1152x over previous
import functools

import jax
import jax.numpy as jnp
from jax import lax
from jax.experimental import pallas as pl
from jax.experimental.pallas import tpu as pltpu
from jax.experimental.pallas import tpu_sc as plsc

B, S, NCAT, V, ED, CD, DM = 4096, 50, 26, 100000, 32, 16, 128
T = B * S
N = T * NCAT
RPT = 128
NW = 32
BLOCKS = N // RPT
BPW = BLOCKS // NW
K = 13
OUTER = BPW // K


def _sc_gather(tables_flat, idx_flat):
    mesh = plsc.VectorSubcoreMesh(core_axis_name="c", subcore_axis_name="s")

    @functools.partial(
        pl.kernel,
        out_type=jax.ShapeDtypeStruct((N, ED), jnp.float32),
        mesh=mesh,
        scratch_types=[
            pltpu.VMEM((K * RPT,), jnp.int32),
            pltpu.VMEM((K * RPT, ED), jnp.float32),
            pltpu.SemaphoreType.DMA,
        ],
        compiler_params=pltpu.CompilerParams(use_tc_tiling_on_sc=False),
    )
    def gather_kernel(tbl_hbm, idx_hbm, out_hbm, idx_v, rows_v, sem):
        wid = lax.axis_index("s") * 2 + lax.axis_index("c")
        wbase = wid * BPW

        def body(g, carry):
            row0 = (wbase + g * K) * RPT
            pltpu.sync_copy(idx_hbm.at[pl.ds(row0, K * RPT)], idx_v)
            copies = []
            for j in range(K):
                copies.append(
                    pltpu.async_copy(
                        tbl_hbm.at[idx_v.at[pl.ds(j * RPT, RPT)]],
                        rows_v.at[pl.ds(j * RPT, RPT)],
                        sem,
                    )
                )
            for c in copies:
                c.wait()
            pltpu.sync_copy(rows_v, out_hbm.at[pl.ds(row0, K * RPT)])
            return carry

        lax.fori_loop(0, OUTER, body, 0)

    return gather_kernel(tables_flat, idx_flat)


def _proj_body(x_ref, n_ref, m_ref, wc_ref, wn_ref, b_ref, o_ref):
    acc = jnp.dot(x_ref[...], wc_ref[...], preferred_element_type=jnp.float32)
    acc += jnp.dot(n_ref[...] * m_ref[...], wn_ref[...],
                   preferred_element_type=jnp.float32)
    o_ref[...] = acc + b_ref[...]


def _tc_project(cat_repr, num_inputs, num_mask, Wc, Wn, b2):
    BT = 2048
    grid = (T // BT,)
    return pl.pallas_call(
        _proj_body,
        grid=grid,
        in_specs=[
            pl.BlockSpec((BT, NCAT * ED), lambda i: (i, 0)),
            pl.BlockSpec((BT, CD), lambda i: (i, 0)),
            pl.BlockSpec((BT, CD), lambda i: (i, 0)),
            pl.BlockSpec((NCAT * ED, DM), lambda i: (0, 0)),
            pl.BlockSpec((CD, DM), lambda i: (0, 0)),
            pl.BlockSpec((1, DM), lambda i: (0, 0)),
        ],
        out_specs=pl.BlockSpec((BT, DM), lambda i: (i, 0)),
        out_shape=jax.ShapeDtypeStruct((T, DM), jnp.float32),
    )(cat_repr, num_inputs, num_mask, Wc, Wn, b2)


def kernel(cat_inputs, cat_mask, num_inputs, num_mask, tables, W, b):
    tables_flat = tables.reshape(NCAT * V, ED)
    offs = (jnp.arange(NCAT, dtype=jnp.int32) * V)[None, None, :]
    idx_flat = (cat_inputs + offs).reshape(N)

    rows = _sc_gather(tables_flat, idx_flat)
    cat_repr = rows.reshape(T, NCAT * ED)

    out = _tc_project(
        cat_repr,
        num_inputs.reshape(T, CD),
        num_mask.reshape(T, CD),
        W[: NCAT * ED],
        W[NCAT * ED :],
        b.reshape(1, DM),
    )
    return out.reshape(B, S, DM)
